# Initial kernel scaffold; baseline (speedup 1.0000x reference)
import jax
import jax.numpy as jnp
from jax.experimental import pallas as pl


def kernel(x, edge_index, W1, b1, W2, b2, W3, b3):
    raise NotImplementedError("write your pallas kernel here")


# speedup vs baseline: 14.7691x; 14.7691x over previous
import functools

import jax
import jax.numpy as jnp
from jax import lax
from jax.experimental import pallas as pl
from jax.experimental.pallas import tpu as pltpu
from jax.experimental.pallas import tpu_sc as plsc

N = 50000
E = 800000
NP = 50176
ZROWS = NP - N
CW = 128
NR = 4
EPAD = 32 * 200 * CW
EROWS = EPAD // CW
ROWS_PER_CORE = EROWS // 2
ROWS_PER_SUB = ROWS_PER_CORE // 16
MK = ROWS_PER_SUB // NR
SR = NP // 16
WB = 16
BB = SR // WB

BN = 2000
GRID = N // BN


def _mesh():
    return plsc.VectorSubcoreMesh(core_axis_name="c", subcore_axis_name="s")


_SC_PARAMS = pltpu.CompilerParams(use_tc_tiling_on_sc=False)


def _deg_kernel():
    @functools.partial(
        pl.kernel,
        mesh=_mesh(),
        compiler_params=_SC_PARAMS,
        out_type=jax.ShapeDtypeStruct((2, NP), jnp.float32),
        scratch_types=[
            pltpu.VMEM((NR, CW), jnp.int32),
            pltpu.VMEM((CW,), jnp.float32),
            pltpu.VMEM((SR,), jnp.float32),
            pltpu.VMEM_SHARED((NP,), jnp.float32),
        ],
    )
    def deg(dst_hbm, dd_hbm, dst_v, ones_v, bounce, acc):
        cid = lax.axis_index("c")
        sid = lax.axis_index("s")
        for i in range(CW // 16):
            ones_v[pl.ds(i * 16, 16)] = jnp.full((16,), 1.0, jnp.float32)

        def zf(i, carry):
            bounce[pl.ds(i * 16, 16)] = jnp.zeros((16,), jnp.float32)
            return carry

        lax.fori_loop(0, SR // 16, zf, 0)
        pltpu.sync_copy(bounce, acc.at[pl.ds(sid * SR, SR)])
        plsc.subcore_barrier()
        base0 = cid * ROWS_PER_CORE + sid * ROWS_PER_SUB

        def chunk(m, carry):
            rb = base0 + m * NR
            pltpu.sync_copy(dst_hbm.at[pl.ds(rb, NR)], dst_v)
            for r in range(NR):
                pltpu.sync_copy(ones_v, acc.at[dst_v.at[r]], add=True)
            return carry

        lax.fori_loop(0, MK, chunk, 0)
        plsc.subcore_barrier()
        pltpu.sync_copy(acc.at[pl.ds(sid * SR, SR)], bounce)
        pltpu.sync_copy(bounce, dd_hbm.at[cid, pl.ds(sid * SR, SR)])

    return deg


def _prop_kernel(ds, ns):
    @functools.partial(
        pl.kernel,
        mesh=_mesh(),
        compiler_params=_SC_PARAMS,
        out_type=[jax.ShapeDtypeStruct((2, NP, ds), jnp.float32)] * ns,
        scratch_types=[
            pltpu.VMEM((NR, CW), jnp.int32),
            pltpu.VMEM((NR, CW), jnp.int32),
            pltpu.VMEM((NR, CW, ds), jnp.float32),
            pltpu.VMEM((BB, ds), jnp.float32),
            pltpu.VMEM_SHARED((NP, ds), jnp.float32),
            pltpu.SemaphoreType.DMA,
        ],
    )
    def prop(src_hbm, dst_hbm, *rest):
        u_refs = rest[:ns]
        out_refs = rest[ns:2 * ns]
        src_v, dst_v, rows_v, bb_v, acc, sem = rest[2 * ns:]
        cid = lax.axis_index("c")
        sid = lax.axis_index("s")
        base0 = cid * ROWS_PER_CORE + sid * ROWS_PER_SUB

        def zf(i, carry):
            for j in range(ds // 16):
                bb_v[i, pl.ds(j * 16, 16)] = jnp.zeros((16,), jnp.float32)
            return carry

        for s in range(ns):
            u = u_refs[s]
            lax.fori_loop(0, BB, zf, 0)
            for b in range(WB):
                pltpu.sync_copy(bb_v, acc.at[pl.ds(sid * SR + b * BB, BB)])
            plsc.subcore_barrier()

            def chunk(m, carry):
                rb = base0 + m * NR
                pltpu.sync_copy(src_hbm.at[pl.ds(rb, NR)], src_v)
                pltpu.sync_copy(dst_hbm.at[pl.ds(rb, NR)], dst_v)
                handles = [
                    pltpu.async_copy(u.at[src_v.at[r]], rows_v.at[r], sem)
                    for r in range(NR)
                ]
                for h in handles:
                    h.wait()
                for r in range(NR):
                    pltpu.sync_copy(rows_v.at[r], acc.at[dst_v.at[r]], add=True)
                return carry

            lax.fori_loop(0, MK, chunk, 0)
            plsc.subcore_barrier()

            o = out_refs[s]
            for b in range(WB):
                pltpu.sync_copy(acc.at[pl.ds(sid * SR + b * BB, BB)], bb_v)
                pltpu.sync_copy(
                    bb_v, o.at[cid, pl.ds(sid * SR + b * BB, BB)])

            plsc.subcore_barrier()

    return prop


def _row(d):
    return pl.BlockSpec((BN, d), lambda i: (i, 0))


def _whole(shape):
    return pl.BlockSpec(shape, lambda i: tuple(0 for _ in shape))


def _tc_b(d0, d1, x):
    def body(d0_r, d1_r, x_r, dinv_o, u0_o):
        deg = d0_r[...] + d1_r[...] + 1.0
        dinv = lax.rsqrt(deg)
        dinv_o[...] = dinv
        u0 = x_r[...] * dinv
        u0_o[...] = jnp.concatenate(
            [u0, jnp.zeros((BN, 4), jnp.float32)], axis=1)

    return pl.pallas_call(
        body,
        grid=(GRID,),
        in_specs=[_row(1), _row(1), _row(12)],
        out_specs=[_row(1), _row(16)],
        out_shape=[
            jax.ShapeDtypeStruct((N, 1), jnp.float32),
            jax.ShapeDtypeStruct((N, 16), jnp.float32),
        ],
    )(d0, d1, x)


def _tc_d(s10, s11, u0, dinv, W1p, b1):
    def body(s10_r, s11_r, u0_r, dinv_r, w_r, b_r, u1a_o, u1b_o):
        t1 = (s10_r[...] + s11_r[...] + u0_r[...]) * dinv_r[...]
        h1 = jnp.dot(t1, w_r[...], preferred_element_type=jnp.float32)
        h1 = jnp.maximum(h1 + b_r[...], 0.0)
        u1 = h1 * dinv_r[...]
        u1a_o[...] = u1[:, :32]
        u1b_o[...] = u1[:, 32:]

    return pl.pallas_call(
        body,
        grid=(GRID,),
        in_specs=[_row(16), _row(16), _row(16), _row(1),
                  _whole((16, 64)), _whole((1, 64))],
        out_specs=[_row(32), _row(32)],
        out_shape=[jax.ShapeDtypeStruct((N, 32), jnp.float32)] * 2,
    )(s10, s11, u0, dinv, W1p, b1)


def _tc_e(s2, u1a, u1b, dinv, W2, b2, W3):
    def body(s200_r, s201_r, s210_r, s211_r, u1a_r, u1b_r, dinv_r,
             w2_r, b2_r, w3_r, u2a_o, u2b_o, u2c_o):
        dv = dinv_r[...]
        qa = (s200_r[...] + s201_r[...] + u1a_r[...]) * dv
        qb = (s210_r[...] + s211_r[...] + u1b_r[...]) * dv
        q = jnp.concatenate([qa, qb], axis=1)
        h2 = jnp.dot(q, w2_r[...], preferred_element_type=jnp.float32)
        h2 = jnp.maximum(h2 + b2_r[...], 0.0)
        g = jnp.dot(h2, w3_r[...], preferred_element_type=jnp.float32)
        u2 = g * dv
        u2a_o[...] = u2[:, :32]
        u2b_o[...] = u2[:, 32:64]
        u2c_o[...] = u2[:, 64:]

    return pl.pallas_call(
        body,
        grid=(GRID,),
        in_specs=[_row(32)] * 4 + [_row(32), _row(32), _row(1),
                  _whole((64, 128)), _whole((1, 128)), _whole((128, 96))],
        out_specs=[_row(32)] * 3,
        out_shape=[jax.ShapeDtypeStruct((N, 32), jnp.float32)] * 3,
    )(*s2, u1a, u1b, dinv, W2, b2, W3)


def _tc_f(s3, u2a, u2b, u2c, dinv, b3):
    def body(s300_r, s301_r, s310_r, s311_r, s320_r, s321_r,
             u2a_r, u2b_r, u2c_r, dinv_r, b3_r, out_o):
        dv = dinv_r[...]
        oa = (s300_r[...] + s301_r[...] + u2a_r[...]) * dv
        ob = (s310_r[...] + s311_r[...] + u2b_r[...]) * dv
        oc = (s320_r[...] + s321_r[...] + u2c_r[...]) * dv
        out_o[...] = jnp.concatenate([oa, ob, oc], axis=1) + b3_r[...]

    return pl.pallas_call(
        body,
        grid=(GRID,),
        in_specs=[_row(32)] * 6 + [_row(32), _row(32), _row(32), _row(1),
                                   _whole((1, 96))],
        out_specs=_row(96),
        out_shape=jax.ShapeDtypeStruct((N, 96), jnp.float32),
    )(*s3, u2a, u2b, u2c, dinv, b3)


def kernel(x, edge_index, W1, b1, W2, b2, W3, b3):
    f32 = jnp.float32
    src = edge_index[0]
    dst = edge_index[1]
    pad = EPAD - E
    pad_rows = N + (jnp.arange(pad, dtype=jnp.int32) % ZROWS)
    src2d = jnp.concatenate([src, pad_rows]).reshape(EROWS, CW)
    dst2d = jnp.concatenate([dst, pad_rows]).reshape(EROWS, CW)

    zpad16 = jnp.zeros((ZROWS, 16), f32)
    zpad32 = jnp.zeros((ZROWS, 32), f32)

    dd = _deg_kernel()(dst2d)
    dinv, u0 = _tc_b(dd[0].reshape(NP, 1), dd[1].reshape(NP, 1), x)

    (s1,) = _prop_kernel(16, 1)(
        src2d, dst2d, jnp.concatenate([u0, zpad16]))
    W1p = jnp.concatenate([W1, jnp.zeros((4, 64), f32)])
    u1a, u1b = _tc_d(s1[0], s1[1], u0, dinv, W1p, b1.reshape(1, 64))

    s2a, s2b = _prop_kernel(32, 2)(
        src2d, dst2d,
        jnp.concatenate([u1a, zpad32]), jnp.concatenate([u1b, zpad32]))
    u2a, u2b, u2c = _tc_e((s2a[0], s2a[1], s2b[0], s2b[1]),
                          u1a, u1b, dinv, W2, b2.reshape(1, 128), W3)

    s3a, s3b, s3c = _prop_kernel(32, 3)(
        src2d, dst2d,
        jnp.concatenate([u2a, zpad32]), jnp.concatenate([u2b, zpad32]),
        jnp.concatenate([u2c, zpad32]))
    out = _tc_f((s3a[0], s3a[1], s3b[0], s3b[1], s3c[0], s3c[1]),
                u2a, u2b, u2c, dinv, b3.reshape(1, 96))
    return out.reshape(N, 8, 12)


# speedup vs baseline: 14.7946x; 1.0017x over previous
import functools

import jax
import jax.numpy as jnp
from jax import lax
from jax.experimental import pallas as pl
from jax.experimental.pallas import tpu as pltpu
from jax.experimental.pallas import tpu_sc as plsc

N = 50000
E = 800000
NP = 50176
ZROWS = NP - N
CW = 128
NR = 4
EPAD = 32 * 200 * CW
EROWS = EPAD // CW
ROWS_PER_CORE = EROWS // 2
ROWS_PER_SUB = ROWS_PER_CORE // 16
MK = ROWS_PER_SUB // NR
SR = NP // 16
WB = 16
BB = SR // WB

BN = 2000
GRID = N // BN


def _mesh():
    return plsc.VectorSubcoreMesh(core_axis_name="c", subcore_axis_name="s")


_SC_PARAMS = pltpu.CompilerParams(use_tc_tiling_on_sc=False)


def _deg_kernel():
    @functools.partial(
        pl.kernel,
        mesh=_mesh(),
        compiler_params=_SC_PARAMS,
        out_type=jax.ShapeDtypeStruct((2, NP), jnp.float32),
        scratch_types=[
            pltpu.VMEM((NR, CW), jnp.int32),
            pltpu.VMEM((CW,), jnp.float32),
            pltpu.VMEM((SR,), jnp.float32),
            pltpu.VMEM_SHARED((NP,), jnp.float32),
        ],
    )
    def deg(dst_hbm, dd_hbm, dst_v, ones_v, bounce, acc):
        cid = lax.axis_index("c")
        sid = lax.axis_index("s")
        for i in range(CW // 16):
            ones_v[pl.ds(i * 16, 16)] = jnp.full((16,), 1.0, jnp.float32)

        def zf(i, carry):
            bounce[pl.ds(i * 16, 16)] = jnp.zeros((16,), jnp.float32)
            return carry

        lax.fori_loop(0, SR // 16, zf, 0)
        pltpu.sync_copy(bounce, acc.at[pl.ds(sid * SR, SR)])
        plsc.subcore_barrier()
        base0 = cid * ROWS_PER_CORE + sid * ROWS_PER_SUB

        def chunk(m, carry):
            rb = base0 + m * NR
            pltpu.sync_copy(dst_hbm.at[pl.ds(rb, NR)], dst_v)
            for r in range(NR):
                pltpu.sync_copy(ones_v, acc.at[dst_v.at[r]], add=True)
            return carry

        lax.fori_loop(0, MK, chunk, 0)
        plsc.subcore_barrier()
        pltpu.sync_copy(acc.at[pl.ds(sid * SR, SR)], bounce)
        pltpu.sync_copy(bounce, dd_hbm.at[cid, pl.ds(sid * SR, SR)])

    return deg


def _prop_kernel(ds, ns):
    MKC = EROWS // 32

    @functools.partial(
        pl.kernel,
        mesh=_mesh(),
        compiler_params=_SC_PARAMS,
        out_type=[jax.ShapeDtypeStruct((2, NP, ds), jnp.float32)] * ns,
        scratch_types=[
            pltpu.VMEM((4, 2, CW), jnp.int32),
            pltpu.VMEM((2, CW, ds), jnp.float32),
            pltpu.VMEM((BB, ds), jnp.float32),
            pltpu.VMEM_SHARED((NP, ds), jnp.float32),
            pltpu.SemaphoreType.DMA,
            pltpu.SemaphoreType.DMA,
            pltpu.SemaphoreType.DMA,
        ],
    )
    def prop(ei_hbm, *rest):
        u_refs = rest[:ns]
        out_refs = rest[ns:2 * ns]
        ei_v, rows_v, bb_v, acc, sem_i, sem_g, sem_s = rest[2 * ns:]
        cid = lax.axis_index("c")
        sid = lax.axis_index("s")
        base0 = cid * ROWS_PER_CORE + sid * ROWS_PER_SUB

        def zf(i, carry):
            for j in range(ds // 16):
                bb_v[i, pl.ds(j * 16, 16)] = jnp.zeros((16,), jnp.float32)
            return carry

        for s in range(ns):
            u = u_refs[s]

            def stage(m, je):
                pltpu.async_copy(ei_hbm.at[base0 + m], ei_v.at[je], sem_i)

            def drain_i(je):
                pltpu.make_async_copy(
                    ei_hbm.at[0], ei_v.at[je], sem_i).wait()

            def gather(je, jb):
                pltpu.async_copy(
                    u.at[ei_v.at[je, 0]], rows_v.at[jb], sem_g)

            def drain_g(jb):
                pltpu.make_async_copy(
                    u.at[pl.ds(0, CW)], rows_v.at[jb], sem_g).wait()

            def scatter(jpe, jp):
                pltpu.async_copy(
                    rows_v.at[jp], acc.at[ei_v.at[jpe, 1]], sem_s, add=True)

            def drain_s(jb):
                pltpu.make_async_copy(
                    u.at[pl.ds(0, CW)], rows_v.at[jb], sem_s).wait()

            def slot(m, j, do_ds, do_g, do_stage):
                jb, jp = j % 2, (j - 1) % 2
                je, jpe, jn = j % 4, (j - 1) % 4, (j + 1) % 4
                if do_ds:
                    drain_s(jb)
                drain_i(je)
                if do_g:
                    drain_g(jp)
                    scatter(jpe, jp)
                gather(je, jb)
                if do_stage:
                    stage(m + 1, jn)

            lax.fori_loop(0, BB, zf, 0)
            for b in range(WB):
                pltpu.sync_copy(bb_v, acc.at[pl.ds(sid * SR + b * BB, BB)])
            plsc.subcore_barrier()

            stage(0, 0)
            for j in range(4):
                slot(j, j, do_ds=False, do_g=(j >= 1), do_stage=True)

            def body(k, carry):
                m = k * 4
                for j in range(4):
                    slot(m + j, j, do_ds=True, do_g=True, do_stage=True)
                return carry

            lax.fori_loop(1, MKC // 4 - 1, body, 0)

            mlast = MKC - 4
            for j in range(4):
                slot(mlast + j, j, do_ds=True, do_g=True,
                     do_stage=(j < 3))
            drain_s((MKC - 2) % 2)
            drain_g((MKC - 1) % 2)
            scatter((MKC - 1) % 4, (MKC - 1) % 2)
            drain_s((MKC - 1) % 2)

            plsc.subcore_barrier()

            o = out_refs[s]
            for b in range(WB):
                pltpu.sync_copy(acc.at[pl.ds(sid * SR + b * BB, BB)], bb_v)
                pltpu.sync_copy(
                    bb_v, o.at[cid, pl.ds(sid * SR + b * BB, BB)])

            plsc.subcore_barrier()

    return prop


def _row(d):
    return pl.BlockSpec((BN, d), lambda i: (i, 0))


def _whole(shape):
    return pl.BlockSpec(shape, lambda i: tuple(0 for _ in shape))


def _tc_b(d0, d1, x):
    def body(d0_r, d1_r, x_r, dinv_o, u0_o):
        deg = d0_r[...] + d1_r[...] + 1.0
        dinv = lax.rsqrt(deg)
        dinv_o[...] = dinv
        u0 = x_r[...] * dinv
        u0_o[...] = jnp.concatenate(
            [u0, jnp.zeros((BN, 4), jnp.float32)], axis=1)

    return pl.pallas_call(
        body,
        grid=(GRID,),
        in_specs=[_row(1), _row(1), _row(12)],
        out_specs=[_row(1), _row(16)],
        out_shape=[
            jax.ShapeDtypeStruct((N, 1), jnp.float32),
            jax.ShapeDtypeStruct((N, 16), jnp.float32),
        ],
    )(d0, d1, x)


def _tc_d(s10, s11, u0, dinv, W1p, b1):
    def body(s10_r, s11_r, u0_r, dinv_r, w_r, b_r, u1a_o, u1b_o):
        t1 = (s10_r[...] + s11_r[...] + u0_r[...]) * dinv_r[...]
        h1 = jnp.dot(t1, w_r[...], preferred_element_type=jnp.float32)
        h1 = jnp.maximum(h1 + b_r[...], 0.0)
        u1 = h1 * dinv_r[...]
        u1a_o[...] = u1[:, :32]
        u1b_o[...] = u1[:, 32:]

    return pl.pallas_call(
        body,
        grid=(GRID,),
        in_specs=[_row(16), _row(16), _row(16), _row(1),
                  _whole((16, 64)), _whole((1, 64))],
        out_specs=[_row(32), _row(32)],
        out_shape=[jax.ShapeDtypeStruct((N, 32), jnp.float32)] * 2,
    )(s10, s11, u0, dinv, W1p, b1)


def _tc_e(s2, u1a, u1b, dinv, W2, b2, W3):
    def body(s200_r, s201_r, s210_r, s211_r, u1a_r, u1b_r, dinv_r,
             w2_r, b2_r, w3_r, u2a_o, u2b_o, u2c_o):
        dv = dinv_r[...]
        qa = (s200_r[...] + s201_r[...] + u1a_r[...]) * dv
        qb = (s210_r[...] + s211_r[...] + u1b_r[...]) * dv
        q = jnp.concatenate([qa, qb], axis=1)
        h2 = jnp.dot(q, w2_r[...], preferred_element_type=jnp.float32)
        h2 = jnp.maximum(h2 + b2_r[...], 0.0)
        g = jnp.dot(h2, w3_r[...], preferred_element_type=jnp.float32)
        u2 = g * dv
        u2a_o[...] = u2[:, :32]
        u2b_o[...] = u2[:, 32:64]
        u2c_o[...] = u2[:, 64:]

    return pl.pallas_call(
        body,
        grid=(GRID,),
        in_specs=[_row(32)] * 4 + [_row(32), _row(32), _row(1),
                  _whole((64, 128)), _whole((1, 128)), _whole((128, 96))],
        out_specs=[_row(32)] * 3,
        out_shape=[jax.ShapeDtypeStruct((N, 32), jnp.float32)] * 3,
    )(*s2, u1a, u1b, dinv, W2, b2, W3)


def _tc_f(s3, u2a, u2b, u2c, dinv, b3):
    def body(s300_r, s301_r, s310_r, s311_r, s320_r, s321_r,
             u2a_r, u2b_r, u2c_r, dinv_r, b3_r, out_o):
        dv = dinv_r[...]
        oa = (s300_r[...] + s301_r[...] + u2a_r[...]) * dv
        ob = (s310_r[...] + s311_r[...] + u2b_r[...]) * dv
        oc = (s320_r[...] + s321_r[...] + u2c_r[...]) * dv
        out_o[...] = jnp.concatenate([oa, ob, oc], axis=1) + b3_r[...]

    return pl.pallas_call(
        body,
        grid=(GRID,),
        in_specs=[_row(32)] * 6 + [_row(32), _row(32), _row(32), _row(1),
                                   _whole((1, 96))],
        out_specs=_row(96),
        out_shape=jax.ShapeDtypeStruct((N, 96), jnp.float32),
    )(*s3, u2a, u2b, u2c, dinv, b3)


def kernel(x, edge_index, W1, b1, W2, b2, W3, b3):
    f32 = jnp.float32
    src = edge_index[0]
    dst = edge_index[1]
    pad = EPAD - E
    pad_rows = N + (jnp.arange(pad, dtype=jnp.int32) % ZROWS)
    src2d = jnp.concatenate([src, pad_rows]).reshape(EROWS, CW)
    dst2d = jnp.concatenate([dst, pad_rows]).reshape(EROWS, CW)
    ei3d = jnp.stack([src2d, dst2d], axis=1)

    zpad16 = jnp.zeros((ZROWS, 16), f32)
    zpad32 = jnp.zeros((ZROWS, 32), f32)

    dd = _deg_kernel()(dst2d)
    dinv, u0 = _tc_b(dd[0].reshape(NP, 1), dd[1].reshape(NP, 1), x)

    (s1,) = _prop_kernel(16, 1)(ei3d, jnp.concatenate([u0, zpad16]))
    W1p = jnp.concatenate([W1, jnp.zeros((4, 64), f32)])
    u1a, u1b = _tc_d(s1[0], s1[1], u0, dinv, W1p, b1.reshape(1, 64))

    s2a, s2b = _prop_kernel(32, 2)(
        ei3d, jnp.concatenate([u1a, zpad32]), jnp.concatenate([u1b, zpad32]))
    u2a, u2b, u2c = _tc_e((s2a[0], s2a[1], s2b[0], s2b[1]),
                          u1a, u1b, dinv, W2, b2.reshape(1, 128), W3)

    s3a, s3b, s3c = _prop_kernel(32, 3)(
        ei3d, jnp.concatenate([u2a, zpad32]), jnp.concatenate([u2b, zpad32]),
        jnp.concatenate([u2c, zpad32]))
    out = _tc_f((s3a[0], s3a[1], s3b[0], s3b[1], s3c[0], s3c[1]),
                u2a, u2b, u2c, dinv, b3.reshape(1, 96))
    return out.reshape(N, 8, 12)


# speedup vs baseline: 17.0456x; 1.1521x over previous
import functools

import jax
import jax.numpy as jnp
from jax import lax
from jax.experimental import pallas as pl
from jax.experimental.pallas import tpu as pltpu
from jax.experimental.pallas import tpu_sc as plsc

N = 50000
E = 800000
NP = 50176
ZROWS = NP - N
CW = 128
NR = 4
EPAD = 32 * 200 * CW
EROWS = EPAD // CW
ROWS_PER_CORE = EROWS // 2
ROWS_PER_SUB = ROWS_PER_CORE // 16
MK = ROWS_PER_SUB // NR
SR = NP // 16
WB = 16
BB = SR // WB

BN = 2000
GRID = N // BN


def _mesh():
    return plsc.VectorSubcoreMesh(core_axis_name="c", subcore_axis_name="s")


_SC_PARAMS = pltpu.CompilerParams(use_tc_tiling_on_sc=False)


def _deg_kernel():
    @functools.partial(
        pl.kernel,
        mesh=_mesh(),
        compiler_params=_SC_PARAMS,
        out_type=jax.ShapeDtypeStruct((2, NP), jnp.float32),
        scratch_types=[
            pltpu.VMEM((NR, CW), jnp.int32),
            pltpu.VMEM((CW,), jnp.float32),
            pltpu.VMEM((SR,), jnp.float32),
            pltpu.VMEM_SHARED((NP,), jnp.float32),
        ],
    )
    def deg(dst_hbm, dd_hbm, dst_v, ones_v, bounce, acc):
        cid = lax.axis_index("c")
        sid = lax.axis_index("s")
        for i in range(CW // 16):
            ones_v[pl.ds(i * 16, 16)] = jnp.full((16,), 1.0, jnp.float32)

        def zf(i, carry):
            bounce[pl.ds(i * 16, 16)] = jnp.zeros((16,), jnp.float32)
            return carry

        lax.fori_loop(0, SR // 16, zf, 0)
        pltpu.sync_copy(bounce, acc.at[pl.ds(sid * SR, SR)])
        plsc.subcore_barrier()
        base0 = cid * ROWS_PER_CORE + sid * ROWS_PER_SUB

        def chunk(m, carry):
            rb = base0 + m * NR
            pltpu.sync_copy(dst_hbm.at[pl.ds(rb, NR)], dst_v)
            for r in range(NR):
                pltpu.sync_copy(ones_v, acc.at[dst_v.at[r]], add=True)
            return carry

        lax.fori_loop(0, MK, chunk, 0)
        plsc.subcore_barrier()
        pltpu.sync_copy(acc.at[pl.ds(sid * SR, SR)], bounce)
        pltpu.sync_copy(bounce, dd_hbm.at[cid, pl.ds(sid * SR, SR)])

    return deg


def _prop_kernel(ds, ns):
    MKC = EROWS // 32

    @functools.partial(
        pl.kernel,
        mesh=_mesh(),
        compiler_params=_SC_PARAMS,
        out_type=[jax.ShapeDtypeStruct((2, NP, ds), jnp.float32)] * ns,
        scratch_types=[
            pltpu.VMEM((4, 2, CW), jnp.int32),
            pltpu.VMEM((2, CW, ds), jnp.float32),
            pltpu.VMEM((BB, ds), jnp.float32),
            pltpu.VMEM_SHARED((NP, ds), jnp.float32),
            pltpu.SemaphoreType.DMA,
            pltpu.SemaphoreType.DMA,
            pltpu.SemaphoreType.DMA,
        ],
    )
    def prop(ei_hbm, *rest):
        u_refs = rest[:ns]
        out_refs = rest[ns:2 * ns]
        ei_v, rows_v, bb_v, acc, sem_i, sem_g, sem_s = rest[2 * ns:]
        cid = lax.axis_index("c")
        sid = lax.axis_index("s")
        base0 = cid * ROWS_PER_CORE + sid * ROWS_PER_SUB

        def zf(i, carry):
            for j in range(ds // 16):
                bb_v[i, pl.ds(j * 16, 16)] = jnp.zeros((16,), jnp.float32)
            return carry

        for s in range(ns):
            u = u_refs[s]

            def stage(m, je):
                pltpu.async_copy(ei_hbm.at[base0 + m], ei_v.at[je], sem_i)

            def drain_i(je):
                pltpu.make_async_copy(
                    ei_hbm.at[0], ei_v.at[je], sem_i).wait()

            def gather(je, jb):
                pltpu.async_copy(
                    u.at[ei_v.at[je, 0]], rows_v.at[jb], sem_g)

            def drain_g(jb):
                pltpu.make_async_copy(
                    u.at[pl.ds(0, CW)], rows_v.at[jb], sem_g).wait()

            def scatter(jpe, jp):
                pltpu.async_copy(
                    rows_v.at[jp], acc.at[ei_v.at[jpe, 1]], sem_s, add=True)

            def drain_s(jb):
                pltpu.make_async_copy(
                    u.at[pl.ds(0, CW)], rows_v.at[jb], sem_s).wait()

            def slot(m, j, do_ds, do_g, do_stage):
                jb, jp = j % 2, (j - 1) % 2
                je, jpe, jn = j % 4, (j - 1) % 4, (j + 1) % 4
                if do_ds:
                    drain_s(jb)
                drain_i(je)
                if do_g:
                    drain_g(jp)
                    scatter(jpe, jp)
                gather(je, jb)
                if do_stage:
                    stage(m + 1, jn)

            lax.fori_loop(0, BB, zf, 0)
            for b in range(WB):
                pltpu.sync_copy(bb_v, acc.at[pl.ds(sid * SR + b * BB, BB)])
            plsc.subcore_barrier()

            stage(0, 0)
            for j in range(4):
                slot(j, j, do_ds=False, do_g=(j >= 1), do_stage=True)

            def body(k, carry):
                m = k * 4
                for j in range(4):
                    slot(m + j, j, do_ds=True, do_g=True, do_stage=True)
                return carry

            lax.fori_loop(1, MKC // 4 - 1, body, 0)

            mlast = MKC - 4
            for j in range(4):
                slot(mlast + j, j, do_ds=True, do_g=True,
                     do_stage=(j < 3))
            drain_s((MKC - 2) % 2)
            drain_g((MKC - 1) % 2)
            scatter((MKC - 1) % 4, (MKC - 1) % 2)
            drain_s((MKC - 1) % 2)

            plsc.subcore_barrier()

            o = out_refs[s]
            for b in range(WB):
                pltpu.sync_copy(acc.at[pl.ds(sid * SR + b * BB, BB)], bb_v)
                pltpu.sync_copy(
                    bb_v, o.at[cid, pl.ds(sid * SR + b * BB, BB)])

            plsc.subcore_barrier()

    return prop


def _row(d):
    return pl.BlockSpec((BN, d), lambda i: (i, 0))


def _whole(shape):
    return pl.BlockSpec(shape, lambda i: tuple(0 for _ in shape))


def _pair(d):
    return pl.BlockSpec((2, BN, d), lambda i: (0, i, 0))


def _tc_b(dd, x):
    def body(dd_r, x_r, dinv_o, u0_o):
        dv = lax.rsqrt(dd_r[0] + dd_r[1] + 1.0)
        dinv_o[...] = dv
        u0 = x_r[...] * dv
        u0_o[...] = jnp.concatenate(
            [u0, jnp.zeros((BN, 4), jnp.float32)], axis=1)

    return pl.pallas_call(
        body,
        grid=(GRID,),
        in_specs=[pl.BlockSpec((2, BN, 1), lambda i: (0, i, 0)), _row(12)],
        out_specs=[_row(1), _row(16)],
        out_shape=[
            jax.ShapeDtypeStruct((N, 1), jnp.float32),
            jax.ShapeDtypeStruct((N, 16), jnp.float32),
        ],
    )(dd, x)


def _tc_d(s1, u0, dinv, W1p, b1):
    def body(s1_r, u0_r, dinv_r, w_r, b_r, u1a_o, u1b_o):
        t1 = (s1_r[0] + s1_r[1] + u0_r[...]) * dinv_r[...]
        h1 = jnp.dot(t1, w_r[...], preferred_element_type=jnp.float32)
        h1 = jnp.maximum(h1 + b_r[...], 0.0)
        u1 = h1 * dinv_r[...]
        u1a_o[...] = u1[:, :32]
        u1b_o[...] = u1[:, 32:]

    return pl.pallas_call(
        body,
        grid=(GRID,),
        in_specs=[_pair(16), _row(16), _row(1),
                  _whole((16, 64)), _whole((1, 64))],
        out_specs=[_row(32), _row(32)],
        out_shape=[jax.ShapeDtypeStruct((N, 32), jnp.float32)] * 2,
    )(s1, u0, dinv, W1p, b1)


def _tc_e(s2a, s2b, u1a, u1b, dinv, W2, b2, W3):
    def body(s2a_r, s2b_r, u1a_r, u1b_r, dinv_r,
             w2_r, b2_r, w3_r, u2a_o, u2b_o, u2c_o):
        dv = dinv_r[...]
        qa = (s2a_r[0] + s2a_r[1] + u1a_r[...]) * dv
        qb = (s2b_r[0] + s2b_r[1] + u1b_r[...]) * dv
        q = jnp.concatenate([qa, qb], axis=1)
        h2 = jnp.dot(q, w2_r[...], preferred_element_type=jnp.float32)
        h2 = jnp.maximum(h2 + b2_r[...], 0.0)
        g = jnp.dot(h2, w3_r[...], preferred_element_type=jnp.float32)
        u2 = g * dv
        u2a_o[...] = u2[:, :32]
        u2b_o[...] = u2[:, 32:64]
        u2c_o[...] = u2[:, 64:]

    return pl.pallas_call(
        body,
        grid=(GRID,),
        in_specs=[_pair(32), _pair(32), _row(32), _row(32), _row(1),
                  _whole((64, 128)), _whole((1, 128)), _whole((128, 96))],
        out_specs=[_row(32)] * 3,
        out_shape=[jax.ShapeDtypeStruct((N, 32), jnp.float32)] * 3,
    )(s2a, s2b, u1a, u1b, dinv, W2, b2, W3)


def _tc_f(s3a, s3b, s3c, u2a, u2b, u2c, dinv, b3):
    def body(s3a_r, s3b_r, s3c_r, u2a_r, u2b_r, u2c_r, dinv_r, b3_r, out_o):
        dv = dinv_r[...]
        oa = (s3a_r[0] + s3a_r[1] + u2a_r[...]) * dv
        ob = (s3b_r[0] + s3b_r[1] + u2b_r[...]) * dv
        oc = (s3c_r[0] + s3c_r[1] + u2c_r[...]) * dv
        out_o[...] = jnp.concatenate([oa, ob, oc], axis=1) + b3_r[...]

    return pl.pallas_call(
        body,
        grid=(GRID,),
        in_specs=[_pair(32)] * 3 + [_row(32), _row(32), _row(32), _row(1),
                                    _whole((1, 96))],
        out_specs=_row(96),
        out_shape=jax.ShapeDtypeStruct((N, 96), jnp.float32),
    )(s3a, s3b, s3c, u2a, u2b, u2c, dinv, b3)


def kernel(x, edge_index, W1, b1, W2, b2, W3, b3):
    f32 = jnp.float32
    src = edge_index[0]
    dst = edge_index[1]
    pad = EPAD - E
    arp = jnp.arange(pad, dtype=jnp.int32)
    src2d = jnp.concatenate([src, arp % 4096]).reshape(EROWS, CW)
    dst2d = jnp.concatenate([dst, N + arp % ZROWS]).reshape(EROWS, CW)
    ei3d = jnp.stack([src2d, dst2d], axis=1)

    dd = _deg_kernel()(dst2d)
    dinv, u0 = _tc_b(dd.reshape(2, NP, 1), x)

    (s1,) = _prop_kernel(16, 1)(ei3d, u0)
    W1p = jnp.concatenate([W1, jnp.zeros((4, 64), f32)])
    u1a, u1b = _tc_d(s1, u0, dinv, W1p, b1.reshape(1, 64))

    s2a, s2b = _prop_kernel(32, 2)(ei3d, u1a, u1b)
    u2a, u2b, u2c = _tc_e(s2a, s2b, u1a, u1b, dinv, W2,
                          b2.reshape(1, 128), W3)

    s3a, s3b, s3c = _prop_kernel(32, 3)(ei3d, u2a, u2b, u2c)
    out = _tc_f(s3a, s3b, s3c, u2a, u2b, u2c, dinv, b3.reshape(1, 96))
    return out.reshape(N, 8, 12)
